# baseline (speedup 1.0000x reference)
import functools

import jax
import jax.numpy as jnp
from jax import lax
from jax.experimental import pallas as pl
from jax.experimental.pallas import tpu as pltpu
from jax.experimental.pallas import tpu_sc as plsc

N_NODES = 10000
N_EDGES = 160000
N_HID = 128

_INTERPRET = False


_NC = 2
_NS = 16
_RSTRIPE = 624
_RLAST = N_NODES - 15 * _RSTRIPE
_SEG_PW = 5120
_SEG_LAST = N_EDGES - 31 * _SEG_PW
_CH = 128


def _sc_segsum(h, src, dst, zrows):
    mesh = plsc.VectorSubcoreMesh(core_axis_name="c", subcore_axis_name="s")

    @functools.partial(
        pl.kernel,
        mesh=mesh,
        out_type=jax.ShapeDtypeStruct((_NC, N_NODES, N_HID), jnp.float32),
        scratch_types=[
            pltpu.VMEM((_CH,), jnp.int32),
            pltpu.VMEM((_CH,), jnp.int32),
            pltpu.VMEM((_CH,), jnp.int32),
            pltpu.VMEM((_CH,), jnp.int32),
            pltpu.VMEM((_CH, N_HID), jnp.float32),
            pltpu.VMEM((_CH, N_HID), jnp.float32),
            pltpu.VMEM_SHARED((N_NODES, N_HID), jnp.float32),
            pltpu.SemaphoreType.DMA,
            pltpu.SemaphoreType.DMA,
            pltpu.SemaphoreType.DMA,
            pltpu.SemaphoreType.DMA,
        ],
    )
    def seg(h_hbm, src_hbm, dst_hbm, z_hbm, out_hbm,
            sidx0, sidx1, didx0, didx1, rows0, rows1, acc,
            gsem0, gsem1, ssem0, ssem1):
        c = lax.axis_index("c")
        s = lax.axis_index("s")
        wid = c * _NS + s
        sidx = (sidx0, sidx1)
        didx = (didx0, didx1)
        rows = (rows0, rows1)
        gsem = (gsem0, gsem1)
        ssem = (ssem0, ssem1)
        r0 = s * _RSTRIPE

        @pl.when(s != 15)
        def _():
            pltpu.sync_copy(z_hbm.at[pl.ds(0, _RSTRIPE)],
                            acc.at[pl.ds(r0, _RSTRIPE)])

        @pl.when(s == 15)
        def _():
            pltpu.sync_copy(z_hbm, acc.at[pl.ds(r0, _RLAST)])

        plsc.subcore_barrier()

        base = wid * _SEG_PW
        nch = jnp.where(wid == 31, _SEG_LAST // _CH, _SEG_PW // _CH)

        def stage_and_gather(j, b):
            off = base + j * _CH
            pltpu.sync_copy(src_hbm.at[pl.ds(off, _CH)], sidx[b])
            pltpu.sync_copy(dst_hbm.at[pl.ds(off, _CH)], didx[b])
            pltpu.async_copy(h_hbm.at[sidx[b]], rows[b], gsem[b])

        stage_and_gather(0, 0)

        def outer(kk, carry):
            for b in (0, 1):
                j = kk * 2 + b
                pltpu.make_async_copy(h_hbm.at[sidx[b]], rows[b],
                                      gsem[b]).wait()
                pltpu.async_copy(rows[b], acc.at[didx[b]], ssem[b], add=True)
                p = j + 1
                nb = 1 - b

                @pl.when(p < nch)
                def _():
                    @pl.when(p >= 2)
                    def _():
                        pltpu.make_async_copy(rows[nb], acc.at[didx[nb]],
                                              ssem[nb]).wait()
                    stage_and_gather(p, nb)
            return carry

        lax.fori_loop(0, lax.div(nch, 2), outer, 0)
        for b in (0, 1):
            pltpu.make_async_copy(rows[b], acc.at[didx[b]], ssem[b]).wait()

        plsc.subcore_barrier()

        @pl.when(s != 15)
        def _():
            pltpu.sync_copy(acc.at[pl.ds(r0, _RSTRIPE)],
                            out_hbm.at[c].at[pl.ds(r0, _RSTRIPE)])

        @pl.when(s == 15)
        def _():
            pltpu.sync_copy(acc.at[pl.ds(r0, _RLAST)],
                            out_hbm.at[c].at[pl.ds(r0, _RLAST)])

    return seg(h, src, dst, zrows)


_EF_PW = 5120
_EF_LAST = N_EDGES - 31 * _EF_PW


def _sc_ef(xflat, src, dst):
    mesh = plsc.VectorSubcoreMesh(core_axis_name="c", subcore_axis_name="s")

    @functools.partial(
        pl.kernel,
        mesh=mesh,
        out_type=jax.ShapeDtypeStruct((4 * N_EDGES,), jnp.float32),
        compiler_params=pltpu.CompilerParams(needs_layout_passes=False),
        scratch_types=[
            pltpu.VMEM((2 * N_NODES,), jnp.float32),
            pltpu.VMEM((_EF_PW,), jnp.int32),
            pltpu.VMEM((_EF_PW,), jnp.int32),
            pltpu.VMEM((4, _EF_PW), jnp.float32),
        ],
    )
    def ef(x_hbm, src_hbm, dst_hbm, out_hbm, xv, sidx, didx, buf):
        c = lax.axis_index("c")
        s = lax.axis_index("s")
        wid = c * _NS + s
        base = wid * _EF_PW
        n = jnp.where(wid == 31, _EF_LAST, _EF_PW)
        pltpu.sync_copy(x_hbm, xv)

        @pl.when(wid != 31)
        def _():
            pltpu.sync_copy(src_hbm.at[pl.ds(base, _EF_PW)], sidx)
            pltpu.sync_copy(dst_hbm.at[pl.ds(base, _EF_PW)], didx)

        @pl.when(wid == 31)
        def _():
            pltpu.sync_copy(src_hbm.at[pl.ds(base, _EF_LAST)],
                            sidx.at[pl.ds(0, _EF_LAST)])
            pltpu.sync_copy(dst_hbm.at[pl.ds(base, _EF_LAST)],
                            didx.at[pl.ds(0, _EF_LAST)])

        def body(i, carry):
            o = i * 16
            sv = sidx[pl.ds(o, 16)] * 2
            dv = didx[pl.ds(o, 16)] * 2
            buf[0, pl.ds(o, 16)] = plsc.load_gather(xv, [sv])
            buf[1, pl.ds(o, 16)] = plsc.load_gather(xv, [sv + 1])
            buf[2, pl.ds(o, 16)] = plsc.load_gather(xv, [dv])
            buf[3, pl.ds(o, 16)] = plsc.load_gather(xv, [dv + 1])
            return carry

        lax.fori_loop(0, n // 16, body, 0)

        @pl.when(wid != 31)
        def _():
            for r in range(4):
                pltpu.sync_copy(buf.at[r].at[pl.ds(0, _EF_PW)],
                                out_hbm.at[pl.ds(r * N_EDGES + base, _EF_PW)])

        @pl.when(wid == 31)
        def _():
            for r in range(4):
                pltpu.sync_copy(buf.at[r].at[pl.ds(0, _EF_LAST)],
                                out_hbm.at[pl.ds(r * N_EDGES + base, _EF_LAST)])

    return ef(xflat, src, dst)


_TN = 2000
_TE = 6400


def _tc_encode(x, W, b):
    def body(x_ref, w_ref, b_ref, o_ref):
        o_ref[...] = jax.nn.relu(
            jnp.dot(x_ref[...], w_ref[...], preferred_element_type=jnp.float32)
            + b_ref[...])

    return pl.pallas_call(
        body,
        grid=(N_NODES // _TN,),
        in_specs=[
            pl.BlockSpec((_TN, 2), lambda i: (i, 0)),
            pl.BlockSpec((2, N_HID), lambda i: (0, 0)),
            pl.BlockSpec((1, N_HID), lambda i: (0, 0)),
        ],
        out_specs=pl.BlockSpec((_TN, N_HID), lambda i: (i, 0)),
        out_shape=jax.ShapeDtypeStruct((N_NODES, N_HID), jnp.float32),
        interpret=_INTERPRET,
    )(x, W, b)


def _tc_layer(h, mp, W, b):
    def body(h_ref, m_ref, w_ref, b_ref, o_ref, g_ref):
        i = pl.program_id(0)
        hm = h_ref[...] + m_ref[0] + m_ref[1]
        hn = jax.nn.relu(
            jnp.dot(hm, w_ref[...], preferred_element_type=jnp.float32)
            + b_ref[...])
        o_ref[...] = hn
        part = jnp.sum(hn, axis=0, keepdims=True)
        part8 = jnp.concatenate([part, jnp.zeros((7, N_HID), jnp.float32)], 0)

        @pl.when(i == 0)
        def _():
            g_ref[...] = jnp.zeros_like(g_ref)

        g_ref[...] += part8

    return pl.pallas_call(
        body,
        grid=(N_NODES // _TN,),
        in_specs=[
            pl.BlockSpec((_TN, N_HID), lambda i: (i, 0)),
            pl.BlockSpec((_NC, _TN, N_HID), lambda i: (0, i, 0)),
            pl.BlockSpec((N_HID, N_HID), lambda i: (0, 0)),
            pl.BlockSpec((1, N_HID), lambda i: (0, 0)),
        ],
        out_specs=(pl.BlockSpec((_TN, N_HID), lambda i: (i, 0)),
                   pl.BlockSpec((8, N_HID), lambda i: (0, 0))),
        out_shape=(jax.ShapeDtypeStruct((N_NODES, N_HID), jnp.float32),
                   jax.ShapeDtypeStruct((8, N_HID), jnp.float32)),
        compiler_params=pltpu.CompilerParams(
            dimension_semantics=("arbitrary",)),
        interpret=_INTERPRET,
    )(h, mp, W, b)


def _tc_reduce(SCB, ef):
    def body(scb_ref, ef_ref, s_ref, mm_ref):
        i = pl.program_id(0)
        scb = scb_ref[...]
        efb = ef_ref[...]

        def pad8_128(v54):
            v = jnp.concatenate([v54, jnp.zeros((3, 4), jnp.float32)], 0)
            return jnp.concatenate([v, jnp.zeros((8, 124), jnp.float32)], 1)

        pe = [scb * efb[c:c + 1, :] for c in range(4)]
        s54 = jnp.concatenate(
            [jnp.sum(p, axis=1, keepdims=True) for p in pe], axis=1)
        inf = jnp.float32(jnp.inf)

        def mn(a, b2):
            va = jnp.where(a == 0, inf, a).min(axis=1, keepdims=True)
            vb = jnp.where(b2 == 0, inf, b2).min(axis=1, keepdims=True)
            return jnp.minimum(va, vb)

        def mx(a, b2):
            va = jnp.where(a == 0, -inf, a).max(axis=1, keepdims=True)
            vb = jnp.where(b2 == 0, -inf, b2).max(axis=1, keepdims=True)
            return jnp.maximum(va, vb)

        cols = jnp.concatenate(
            [mn(pe[0], pe[2]), mx(pe[0], pe[2]),
             mn(pe[1], pe[3]), mx(pe[1], pe[3])], axis=1)
        new = pad8_128(cols)
        colid = lax.broadcasted_iota(jnp.int32, (8, 128), 1)
        is_min = (colid == 0) | (colid == 2)

        @pl.when(i == 0)
        def _():
            s_ref[...] = jnp.zeros_like(s_ref)
            mm_ref[...] = jnp.where(is_min, inf, -inf)

        s_ref[...] += pad8_128(s54)
        old = mm_ref[...]
        mm_ref[...] = jnp.where(is_min, jnp.minimum(old, new),
                                jnp.maximum(old, new))

    return pl.pallas_call(
        body,
        grid=(N_EDGES // _TE,),
        in_specs=[
            pl.BlockSpec((5, _TE), lambda i: (0, i)),
            pl.BlockSpec((4, _TE), lambda i: (0, i)),
        ],
        out_specs=(pl.BlockSpec((8, 128), lambda i: (0, 0)),
                   pl.BlockSpec((8, 128), lambda i: (0, 0))),
        out_shape=(jax.ShapeDtypeStruct((8, 128), jnp.float32),
                   jax.ShapeDtypeStruct((8, 128), jnp.float32)),
        compiler_params=pltpu.CompilerParams(
            dimension_semantics=("arbitrary",)),
        interpret=_INTERPRET,
    )(SCB, ef)


def _tc_prep(s_pad, W1a, b1a, W1b, b1b, W2aU, b2a, W2b, W4a, b2b, b4a):
    def body(s_ref, w1a, b1a_, w1b, b1b_, w2au, b2a_, w2b, w4a, b2b_, b4a_,
             at_ref, w24_ref, c24_ref):
        s8 = s_ref[...][:, 0:4]
        h1 = jax.nn.relu(jnp.dot(s8, w1a[...],
                                 preferred_element_type=jnp.float32, precision=lax.Precision.HIGHEST) + b1a_[...])
        e1 = jnp.dot(h1, w1b[...], preferred_element_type=jnp.float32, precision=lax.Precision.HIGHEST) + b1b_[...]
        A8 = jnp.dot(e1, w2au[...], preferred_element_type=jnp.float32, precision=lax.Precision.HIGHEST) + b2a_[...]
        at_ref[...] = jnp.transpose(A8)
        W24v = jnp.dot(w2b[...], w4a[...],
                       preferred_element_type=jnp.float32,
                       precision=lax.Precision.HIGHEST)
        w24_ref[...] = W24v
        c24 = 5.0 * jnp.dot(b2b_[...], w4a[...],
                            preferred_element_type=jnp.float32, precision=lax.Precision.HIGHEST) + b4a_[...]
        rowid = lax.broadcasted_iota(jnp.int32, (8, 128), 0)
        reluA = jnp.where(rowid < 5, jax.nn.relu(A8), 0.0)
        R0 = jnp.sum(reluA, axis=0, keepdims=True)
        z0 = jnp.dot(R0, W24v,
                     preferred_element_type=jnp.float32,
                     precision=lax.Precision.HIGHEST) + c24
        z0p = jnp.concatenate([z0, jnp.zeros((7, 128), jnp.float32)], 0)
        c24_ref[...] = jnp.transpose(z0p)

    full = lambda shp: pl.BlockSpec(shp, lambda: (0,) * len(shp))
    return pl.pallas_call(
        body,
        in_specs=[full((8, 128)), full((4, 64)), full((1, 64)),
                  full((64, 64)), full((1, 64)), full((64, 128)),
                  full((1, 128)), full((128, 128)), full((128, 128)),
                  full((1, 128)), full((1, 128))],
        out_specs=(full((128, 8)), full((128, 128)), full((128, 8))),
        out_shape=(jax.ShapeDtypeStruct((128, 8), jnp.float32),
                   jax.ShapeDtypeStruct((128, 128), jnp.float32),
                   jax.ShapeDtypeStruct((128, 8), jnp.float32)),
        interpret=_INTERPRET,
    )(s_pad, W1a, b1a, W1b, b1b, W2aU, b2a, W2b, W4a, b2b, b4a)


def _tc_edge(SCB, ef, AT, W2a4, W24, z0T):
    def body(scb_ref, ef_ref, at_ref, w2a4_ref, w24_ref, z0_ref, t_ref):
        i = pl.program_id(0)
        scb = scb_ref[...]
        efb = ef_ref[...]
        vT = lax.dot_general(w2a4_ref[...].astype(jnp.bfloat16),
                             efb.astype(jnp.bfloat16),
                             (((0,), (0,)), ((), ())),
                             preferred_element_type=jnp.float32)
        at = at_ref[...]
        minA = jnp.minimum(at, 0.0)
        dT = jnp.zeros_like(vT)
        for b in range(5):
            dT += jnp.maximum(scb[b:b + 1, :] * vT, -at[:, b:b + 1]) \
                + minA[:, b:b + 1]
        tT = jax.nn.relu(
            lax.dot_general(w24_ref[...].astype(jnp.bfloat16),
                            dT.astype(jnp.bfloat16),
                            (((0,), (0,)), ((), ())),
                            preferred_element_type=jnp.float32)
            + z0_ref[...][:, 0:1])
        part = jnp.sum(tT, axis=1, keepdims=True)
        part8 = jnp.concatenate([part, jnp.zeros((128, 7), jnp.float32)], 1)

        @pl.when(i == 0)
        def _():
            t_ref[...] = jnp.zeros_like(t_ref)

        t_ref[...] += part8

    return pl.pallas_call(
        body,
        grid=(N_EDGES // _TE,),
        in_specs=[
            pl.BlockSpec((5, _TE), lambda i: (0, i)),
            pl.BlockSpec((4, _TE), lambda i: (0, i)),
            pl.BlockSpec((128, 8), lambda i: (0, 0)),
            pl.BlockSpec((4, 128), lambda i: (0, 0)),
            pl.BlockSpec((128, 128), lambda i: (0, 0)),
            pl.BlockSpec((128, 8), lambda i: (0, 0)),
        ],
        out_specs=pl.BlockSpec((128, 8), lambda i: (0, 0)),
        out_shape=jax.ShapeDtypeStruct((128, 8), jnp.float32),
        compiler_params=pltpu.CompilerParams(
            dimension_semantics=("arbitrary",)),
        interpret=_INTERPRET,
    )(SCB, ef, AT, W2a4, W24, z0T)


def _tc_final(gsum, T, mm, W4b, b4b, finaW, finab, finbW, finbb,
              feataW, featab, featbW, featbb, featcW, featcb):
    def body(g_ref, t_ref, mm_ref, w4b, b4b_, faw, fab, fbw, fbb,
             qaw, qab, qbw, qbb, qcw, qcb, o_ref):
        g = jnp.sum(g_ref[...], axis=0, keepdims=True) / N_NODES
        Tt = jnp.transpose(t_ref[...])
        Trow = Tt[0:1, :]
        L1sum = jnp.dot(Trow, w4b[...],
                        preferred_element_type=jnp.float32, precision=lax.Precision.HIGHEST) + N_EDGES * b4b_[...]
        cat = jnp.concatenate([g, L1sum], axis=1)
        m1 = jax.nn.relu(jnp.dot(cat, faw[...],
                                 preferred_element_type=jnp.float32, precision=lax.Precision.HIGHEST) + fab[...])
        main = jnp.dot(m1, fbw[...],
                       preferred_element_type=jnp.float32, precision=lax.Precision.HIGHEST) + fbb[...]
        mmt = jnp.transpose(mm_ref[...])
        L1f = jnp.concatenate([mmt[0:1, 0:5], mmt[1:2, 0:5],
                               mmt[2:3, 0:5], mmt[3:4, 0:5]], axis=1)
        f1 = jax.nn.relu(jnp.dot(L1f, qaw[...],
                                 preferred_element_type=jnp.float32, precision=lax.Precision.HIGHEST) + qab[...])
        f2 = jax.nn.relu(jnp.dot(f1, qbw[...],
                                 preferred_element_type=jnp.float32, precision=lax.Precision.HIGHEST) + qbb[...])
        feat = jnp.dot(f2, qcw[...],
                       preferred_element_type=jnp.float32, precision=lax.Precision.HIGHEST) + qcb[...]
        o_ref[...] = main + feat

    full = lambda shp: pl.BlockSpec(shp, lambda: (0,) * len(shp))
    return pl.pallas_call(
        body,
        in_specs=[full((8, 128)), full((128, 8)), full((8, 128)),
                  full((128, 128)), full((1, 128)),
                  full((256, 128)), full((1, 128)),
                  full((128, 64)), full((1, 64)),
                  full((20, 128)), full((1, 128)),
                  full((128, 128)), full((1, 128)),
                  full((128, 64)), full((1, 64))],
        out_specs=full((1, 64)),
        out_shape=jax.ShapeDtypeStruct((1, 64), jnp.float32),
        interpret=_INTERPRET,
    )(gsum, T, mm, W4b, b4b, finaW, finab, finbW, finbb,
      feataW, featab, featbW, featbb, featcW, featcb)


def kernel(x, edge_index, SCB, params):
    p = params
    src = edge_index[0].astype(jnp.int32)
    dst = edge_index[1].astype(jnp.int32)
    row = lambda b: b.reshape(1, -1)

    zrows = jnp.zeros((_RLAST, N_HID), jnp.float32)
    h = _tc_encode(x, p["gnn_enc"]["W"], row(p["gnn_enc"]["b"]))
    gsum = None
    for lp in p["gnn_layers"]:
        mp = _sc_segsum(h, src, dst, zrows)
        h, gsum = _tc_layer(h, mp, lp["W"], row(lp["b"]))

    ef = _sc_ef(x.reshape(-1), src, dst).reshape(4, N_EDGES)

    s_pad, mm = _tc_reduce(SCB, ef)
    AT, W24, c24T = _tc_prep(
        s_pad, p["scb1_a"]["W"], row(p["scb1_a"]["b"]),
        p["scb1_b"]["W"], row(p["scb1_b"]["b"]),
        p["scb2_a"]["W"][:64], row(p["scb2_a"]["b"]),
        p["scb2_b"]["W"], p["scb4_a"]["W"],
        row(p["scb2_b"]["b"]), row(p["scb4_a"]["b"]))
    T = _tc_edge(SCB, ef, AT, p["scb2_a"]["W"][64:], W24, c24T)

    return _tc_final(
        gsum, T, mm, p["scb4_b"]["W"], row(p["scb4_b"]["b"]),
        p["fin_a"]["W"], row(p["fin_a"]["b"]),
        p["fin_b"]["W"], row(p["fin_b"]["b"]),
        p["feat_a"]["W"], row(p["feat_a"]["b"]),
        p["feat_b"]["W"], row(p["feat_b"]["b"]),
        p["feat_c"]["W"], row(p["feat_c"]["b"]))


# speedup vs baseline: 1.1724x; 1.1724x over previous
import functools

import jax
import jax.numpy as jnp
from jax import lax
from jax.experimental import pallas as pl
from jax.experimental.pallas import tpu as pltpu
from jax.experimental.pallas import tpu_sc as plsc

N_NODES = 10000
N_EDGES = 160000
N_HID = 128

_INTERPRET = False


_NC = 2
_NS = 16
_RSTRIPE = 624
_RLAST = N_NODES - 15 * _RSTRIPE
_SEG_PW = 5120
_SEG_LAST = N_EDGES - 31 * _SEG_PW
_CH = 128


_NCH = _SEG_PW // _CH


def _sc_segsum(h, src2, dst2, zrows):
    mesh = plsc.VectorSubcoreMesh(core_axis_name="c", subcore_axis_name="s")

    @functools.partial(
        pl.kernel,
        mesh=mesh,
        out_type=jax.ShapeDtypeStruct((_NC, N_NODES, N_HID), jnp.float32),
        scratch_types=[
            pltpu.VMEM((_NCH, _CH), jnp.int32),
            pltpu.VMEM((_NCH, _CH), jnp.int32),
            pltpu.VMEM((_CH, N_HID), jnp.float32),
            pltpu.VMEM((_CH, N_HID), jnp.float32),
            pltpu.VMEM_SHARED((N_NODES, N_HID), jnp.float32),
            pltpu.SemaphoreType.DMA,
            pltpu.SemaphoreType.DMA,
            pltpu.SemaphoreType.DMA,
            pltpu.SemaphoreType.DMA,
        ],
    )
    def seg(h_hbm, src_hbm, dst_hbm, z_hbm, out_hbm,
            sidx2, didx2, rows0, rows1, acc,
            gsem0, gsem1, ssem0, ssem1):
        c = lax.axis_index("c")
        s = lax.axis_index("s")
        wid = c * _NS + s
        rows = (rows0, rows1)
        gsem = (gsem0, gsem1)
        ssem = (ssem0, ssem1)
        r0 = s * _RSTRIPE

        @pl.when(s != 15)
        def _():
            pltpu.sync_copy(z_hbm.at[pl.ds(0, _RSTRIPE)],
                            acc.at[pl.ds(r0, _RSTRIPE)])

        @pl.when(s == 15)
        def _():
            pltpu.sync_copy(z_hbm, acc.at[pl.ds(r0, _RLAST)])

        nch = jnp.where(wid == 31, _SEG_LAST // _CH, _NCH)
        pltpu.sync_copy(src_hbm.at[pl.ds(wid * _NCH, _NCH)], sidx2)
        pltpu.sync_copy(dst_hbm.at[pl.ds(wid * _NCH, _NCH)], didx2)

        plsc.subcore_barrier()

        def g_start(j, b):
            pltpu.async_copy(h_hbm.at[sidx2.at[j]], rows[b], gsem[b])

        def g_wait(b):
            pltpu.make_async_copy(h_hbm.at[sidx2.at[0]], rows[b],
                                  gsem[b]).wait()

        def s_start(j, b):
            pltpu.async_copy(rows[b], acc.at[didx2.at[j]], ssem[b], add=True)

        def s_wait(b):
            pltpu.make_async_copy(rows[b], acc.at[didx2.at[0]],
                                  ssem[b]).wait()

        g_start(0, 0)

        def outer(kk, carry):
            for b in (0, 1):
                j = kk * 2 + b
                g_wait(b)
                s_start(j, b)
                p = j + 1
                nb = 1 - b

                @pl.when(p < nch)
                def _():
                    @pl.when(p >= 2)
                    def _():
                        s_wait(nb)
                    g_start(p, nb)
            return carry

        lax.fori_loop(0, lax.div(nch, 2), outer, 0)
        for b in (0, 1):
            s_wait(b)

        plsc.subcore_barrier()

        @pl.when(s != 15)
        def _():
            pltpu.sync_copy(acc.at[pl.ds(r0, _RSTRIPE)],
                            out_hbm.at[c].at[pl.ds(r0, _RSTRIPE)])

        @pl.when(s == 15)
        def _():
            pltpu.sync_copy(acc.at[pl.ds(r0, _RLAST)],
                            out_hbm.at[c].at[pl.ds(r0, _RLAST)])

    return seg(h, src2, dst2, zrows)


_EF_PW = 5120
_EF_LAST = N_EDGES - 31 * _EF_PW


def _sc_ef(xflat, src, dst):
    mesh = plsc.VectorSubcoreMesh(core_axis_name="c", subcore_axis_name="s")

    @functools.partial(
        pl.kernel,
        mesh=mesh,
        out_type=jax.ShapeDtypeStruct((4 * N_EDGES,), jnp.float32),
        compiler_params=pltpu.CompilerParams(needs_layout_passes=False),
        scratch_types=[
            pltpu.VMEM((2 * N_NODES,), jnp.float32),
            pltpu.VMEM((_EF_PW,), jnp.int32),
            pltpu.VMEM((_EF_PW,), jnp.int32),
            pltpu.VMEM((4, _EF_PW), jnp.float32),
        ],
    )
    def ef(x_hbm, src_hbm, dst_hbm, out_hbm, xv, sidx, didx, buf):
        c = lax.axis_index("c")
        s = lax.axis_index("s")
        wid = c * _NS + s
        base = wid * _EF_PW
        n = jnp.where(wid == 31, _EF_LAST, _EF_PW)
        pltpu.sync_copy(x_hbm, xv)

        @pl.when(wid != 31)
        def _():
            pltpu.sync_copy(src_hbm.at[pl.ds(base, _EF_PW)], sidx)
            pltpu.sync_copy(dst_hbm.at[pl.ds(base, _EF_PW)], didx)

        @pl.when(wid == 31)
        def _():
            pltpu.sync_copy(src_hbm.at[pl.ds(base, _EF_LAST)],
                            sidx.at[pl.ds(0, _EF_LAST)])
            pltpu.sync_copy(dst_hbm.at[pl.ds(base, _EF_LAST)],
                            didx.at[pl.ds(0, _EF_LAST)])

        def body(i, carry):
            o = i * 16
            sv = sidx[pl.ds(o, 16)] * 2
            dv = didx[pl.ds(o, 16)] * 2
            buf[0, pl.ds(o, 16)] = plsc.load_gather(xv, [sv])
            buf[1, pl.ds(o, 16)] = plsc.load_gather(xv, [sv + 1])
            buf[2, pl.ds(o, 16)] = plsc.load_gather(xv, [dv])
            buf[3, pl.ds(o, 16)] = plsc.load_gather(xv, [dv + 1])
            return carry

        lax.fori_loop(0, n // 16, body, 0)

        @pl.when(wid != 31)
        def _():
            for r in range(4):
                pltpu.sync_copy(buf.at[r].at[pl.ds(0, _EF_PW)],
                                out_hbm.at[pl.ds(r * N_EDGES + base, _EF_PW)])

        @pl.when(wid == 31)
        def _():
            for r in range(4):
                pltpu.sync_copy(buf.at[r].at[pl.ds(0, _EF_LAST)],
                                out_hbm.at[pl.ds(r * N_EDGES + base, _EF_LAST)])

    return ef(xflat, src, dst)


_TN = 2000
_TE = 6400


def _tc_encode(x, W, b):
    def body(x_ref, w_ref, b_ref, o_ref):
        o_ref[...] = jax.nn.relu(
            jnp.dot(x_ref[...], w_ref[...], preferred_element_type=jnp.float32)
            + b_ref[...])

    return pl.pallas_call(
        body,
        grid=(N_NODES // _TN,),
        in_specs=[
            pl.BlockSpec((_TN, 2), lambda i: (i, 0)),
            pl.BlockSpec((2, N_HID), lambda i: (0, 0)),
            pl.BlockSpec((1, N_HID), lambda i: (0, 0)),
        ],
        out_specs=pl.BlockSpec((_TN, N_HID), lambda i: (i, 0)),
        out_shape=jax.ShapeDtypeStruct((N_NODES, N_HID), jnp.float32),
        interpret=_INTERPRET,
    )(x, W, b)


def _tc_layer(h, mp, W, b):
    def body(h_ref, m_ref, w_ref, b_ref, o_ref, g_ref):
        i = pl.program_id(0)
        hm = h_ref[...] + m_ref[0] + m_ref[1]
        hn = jax.nn.relu(
            jnp.dot(hm, w_ref[...], preferred_element_type=jnp.float32)
            + b_ref[...])
        o_ref[...] = hn
        part = jnp.sum(hn, axis=0, keepdims=True)
        part8 = jnp.concatenate([part, jnp.zeros((7, N_HID), jnp.float32)], 0)

        @pl.when(i == 0)
        def _():
            g_ref[...] = jnp.zeros_like(g_ref)

        g_ref[...] += part8

    return pl.pallas_call(
        body,
        grid=(N_NODES // _TN,),
        in_specs=[
            pl.BlockSpec((_TN, N_HID), lambda i: (i, 0)),
            pl.BlockSpec((_NC, _TN, N_HID), lambda i: (0, i, 0)),
            pl.BlockSpec((N_HID, N_HID), lambda i: (0, 0)),
            pl.BlockSpec((1, N_HID), lambda i: (0, 0)),
        ],
        out_specs=(pl.BlockSpec((_TN, N_HID), lambda i: (i, 0)),
                   pl.BlockSpec((8, N_HID), lambda i: (0, 0))),
        out_shape=(jax.ShapeDtypeStruct((N_NODES, N_HID), jnp.float32),
                   jax.ShapeDtypeStruct((8, N_HID), jnp.float32)),
        compiler_params=pltpu.CompilerParams(
            dimension_semantics=("arbitrary",)),
        interpret=_INTERPRET,
    )(h, mp, W, b)


def _tc_reduce(SCB, ef):
    def body(scb_ref, ef_ref, s_ref, mm_ref):
        i = pl.program_id(0)
        scb = scb_ref[...]
        efb = ef_ref[...]

        def pad8_128(v54):
            v = jnp.concatenate([v54, jnp.zeros((3, 4), jnp.float32)], 0)
            return jnp.concatenate([v, jnp.zeros((8, 124), jnp.float32)], 1)

        pe = [scb * efb[c:c + 1, :] for c in range(4)]
        s54 = jnp.concatenate(
            [jnp.sum(p, axis=1, keepdims=True) for p in pe], axis=1)
        inf = jnp.float32(jnp.inf)

        def mn(a, b2):
            va = jnp.where(a == 0, inf, a).min(axis=1, keepdims=True)
            vb = jnp.where(b2 == 0, inf, b2).min(axis=1, keepdims=True)
            return jnp.minimum(va, vb)

        def mx(a, b2):
            va = jnp.where(a == 0, -inf, a).max(axis=1, keepdims=True)
            vb = jnp.where(b2 == 0, -inf, b2).max(axis=1, keepdims=True)
            return jnp.maximum(va, vb)

        cols = jnp.concatenate(
            [mn(pe[0], pe[2]), mx(pe[0], pe[2]),
             mn(pe[1], pe[3]), mx(pe[1], pe[3])], axis=1)
        new = pad8_128(cols)
        colid = lax.broadcasted_iota(jnp.int32, (8, 128), 1)
        is_min = (colid == 0) | (colid == 2)

        @pl.when(i == 0)
        def _():
            s_ref[...] = jnp.zeros_like(s_ref)
            mm_ref[...] = jnp.where(is_min, inf, -inf)

        s_ref[...] += pad8_128(s54)
        old = mm_ref[...]
        mm_ref[...] = jnp.where(is_min, jnp.minimum(old, new),
                                jnp.maximum(old, new))

    return pl.pallas_call(
        body,
        grid=(N_EDGES // _TE,),
        in_specs=[
            pl.BlockSpec((5, _TE), lambda i: (0, i)),
            pl.BlockSpec((4, _TE), lambda i: (0, i)),
        ],
        out_specs=(pl.BlockSpec((8, 128), lambda i: (0, 0)),
                   pl.BlockSpec((8, 128), lambda i: (0, 0))),
        out_shape=(jax.ShapeDtypeStruct((8, 128), jnp.float32),
                   jax.ShapeDtypeStruct((8, 128), jnp.float32)),
        compiler_params=pltpu.CompilerParams(
            dimension_semantics=("arbitrary",)),
        interpret=_INTERPRET,
    )(SCB, ef)


def _tc_prep(s_pad, W1a, b1a, W1b, b1b, W2aU, b2a, W2b, W4a, b2b, b4a):
    def body(s_ref, w1a, b1a_, w1b, b1b_, w2au, b2a_, w2b, w4a, b2b_, b4a_,
             at_ref, w24_ref, c24_ref):
        s8 = s_ref[...][:, 0:4]
        h1 = jax.nn.relu(jnp.dot(s8, w1a[...],
                                 preferred_element_type=jnp.float32, precision=lax.Precision.HIGHEST) + b1a_[...])
        e1 = jnp.dot(h1, w1b[...], preferred_element_type=jnp.float32, precision=lax.Precision.HIGHEST) + b1b_[...]
        A8 = jnp.dot(e1, w2au[...], preferred_element_type=jnp.float32, precision=lax.Precision.HIGHEST) + b2a_[...]
        at_ref[...] = jnp.transpose(A8)
        W24v = jnp.dot(w2b[...], w4a[...],
                       preferred_element_type=jnp.float32,
                       precision=lax.Precision.HIGHEST)
        w24_ref[...] = W24v
        c24 = 5.0 * jnp.dot(b2b_[...], w4a[...],
                            preferred_element_type=jnp.float32, precision=lax.Precision.HIGHEST) + b4a_[...]
        rowid = lax.broadcasted_iota(jnp.int32, (8, 128), 0)
        reluA = jnp.where(rowid < 5, jax.nn.relu(A8), 0.0)
        R0 = jnp.sum(reluA, axis=0, keepdims=True)
        z0 = jnp.dot(R0, W24v,
                     preferred_element_type=jnp.float32,
                     precision=lax.Precision.HIGHEST) + c24
        z0p = jnp.concatenate([z0, jnp.zeros((7, 128), jnp.float32)], 0)
        c24_ref[...] = jnp.transpose(z0p)

    full = lambda shp: pl.BlockSpec(shp, lambda: (0,) * len(shp))
    return pl.pallas_call(
        body,
        in_specs=[full((8, 128)), full((4, 64)), full((1, 64)),
                  full((64, 64)), full((1, 64)), full((64, 128)),
                  full((1, 128)), full((128, 128)), full((128, 128)),
                  full((1, 128)), full((1, 128))],
        out_specs=(full((128, 8)), full((128, 128)), full((128, 8))),
        out_shape=(jax.ShapeDtypeStruct((128, 8), jnp.float32),
                   jax.ShapeDtypeStruct((128, 128), jnp.float32),
                   jax.ShapeDtypeStruct((128, 8), jnp.float32)),
        interpret=_INTERPRET,
    )(s_pad, W1a, b1a, W1b, b1b, W2aU, b2a, W2b, W4a, b2b, b4a)


def _tc_edge(SCB, ef, AT, W2a4, W24, z0T):
    def body(scb_ref, ef_ref, at_ref, w2a4_ref, w24_ref, z0_ref, t_ref):
        i = pl.program_id(0)
        scb = scb_ref[...]
        efb = ef_ref[...]
        vT = lax.dot_general(w2a4_ref[...].astype(jnp.bfloat16),
                             efb.astype(jnp.bfloat16),
                             (((0,), (0,)), ((), ())),
                             preferred_element_type=jnp.float32)
        at = at_ref[...]
        minA = jnp.minimum(at, 0.0)
        dT = jnp.zeros_like(vT)
        for b in range(5):
            dT += jnp.maximum(scb[b:b + 1, :] * vT, -at[:, b:b + 1]) \
                + minA[:, b:b + 1]
        tT = jax.nn.relu(
            lax.dot_general(w24_ref[...].astype(jnp.bfloat16),
                            dT.astype(jnp.bfloat16),
                            (((0,), (0,)), ((), ())),
                            preferred_element_type=jnp.float32)
            + z0_ref[...][:, 0:1])
        part = jnp.sum(tT, axis=1, keepdims=True)
        part8 = jnp.concatenate([part, jnp.zeros((128, 7), jnp.float32)], 1)

        @pl.when(i == 0)
        def _():
            t_ref[...] = jnp.zeros_like(t_ref)

        t_ref[...] += part8

    return pl.pallas_call(
        body,
        grid=(N_EDGES // _TE,),
        in_specs=[
            pl.BlockSpec((5, _TE), lambda i: (0, i)),
            pl.BlockSpec((4, _TE), lambda i: (0, i)),
            pl.BlockSpec((128, 8), lambda i: (0, 0)),
            pl.BlockSpec((4, 128), lambda i: (0, 0)),
            pl.BlockSpec((128, 128), lambda i: (0, 0)),
            pl.BlockSpec((128, 8), lambda i: (0, 0)),
        ],
        out_specs=pl.BlockSpec((128, 8), lambda i: (0, 0)),
        out_shape=jax.ShapeDtypeStruct((128, 8), jnp.float32),
        compiler_params=pltpu.CompilerParams(
            dimension_semantics=("arbitrary",)),
        interpret=_INTERPRET,
    )(SCB, ef, AT, W2a4, W24, z0T)


def _tc_final(gsum, T, mm, W4b, b4b, finaW, finab, finbW, finbb,
              feataW, featab, featbW, featbb, featcW, featcb):
    def body(g_ref, t_ref, mm_ref, w4b, b4b_, faw, fab, fbw, fbb,
             qaw, qab, qbw, qbb, qcw, qcb, o_ref):
        g = jnp.sum(g_ref[...], axis=0, keepdims=True) / N_NODES
        Tt = jnp.transpose(t_ref[...])
        Trow = Tt[0:1, :]
        L1sum = jnp.dot(Trow, w4b[...],
                        preferred_element_type=jnp.float32, precision=lax.Precision.HIGHEST) + N_EDGES * b4b_[...]
        cat = jnp.concatenate([g, L1sum], axis=1)
        m1 = jax.nn.relu(jnp.dot(cat, faw[...],
                                 preferred_element_type=jnp.float32, precision=lax.Precision.HIGHEST) + fab[...])
        main = jnp.dot(m1, fbw[...],
                       preferred_element_type=jnp.float32, precision=lax.Precision.HIGHEST) + fbb[...]
        mmt = jnp.transpose(mm_ref[...])
        L1f = jnp.concatenate([mmt[0:1, 0:5], mmt[1:2, 0:5],
                               mmt[2:3, 0:5], mmt[3:4, 0:5]], axis=1)
        f1 = jax.nn.relu(jnp.dot(L1f, qaw[...],
                                 preferred_element_type=jnp.float32, precision=lax.Precision.HIGHEST) + qab[...])
        f2 = jax.nn.relu(jnp.dot(f1, qbw[...],
                                 preferred_element_type=jnp.float32, precision=lax.Precision.HIGHEST) + qbb[...])
        feat = jnp.dot(f2, qcw[...],
                       preferred_element_type=jnp.float32, precision=lax.Precision.HIGHEST) + qcb[...]
        o_ref[...] = main + feat

    full = lambda shp: pl.BlockSpec(shp, lambda: (0,) * len(shp))
    return pl.pallas_call(
        body,
        in_specs=[full((8, 128)), full((128, 8)), full((8, 128)),
                  full((128, 128)), full((1, 128)),
                  full((256, 128)), full((1, 128)),
                  full((128, 64)), full((1, 64)),
                  full((20, 128)), full((1, 128)),
                  full((128, 128)), full((1, 128)),
                  full((128, 64)), full((1, 64))],
        out_specs=full((1, 64)),
        out_shape=jax.ShapeDtypeStruct((1, 64), jnp.float32),
        interpret=_INTERPRET,
    )(gsum, T, mm, W4b, b4b, finaW, finab, finbW, finbb,
      feataW, featab, featbW, featbb, featcW, featcb)


def kernel(x, edge_index, SCB, params):
    p = params
    src = edge_index[0].astype(jnp.int32)
    dst = edge_index[1].astype(jnp.int32)
    row = lambda b: b.reshape(1, -1)

    zrows = jnp.zeros((_RLAST, N_HID), jnp.float32)
    pad = jnp.zeros((_NC * _NS * _NCH * _CH - N_EDGES,), jnp.int32)
    src2 = jnp.concatenate([src, pad]).reshape(-1, _CH)
    dst2 = jnp.concatenate([dst, pad]).reshape(-1, _CH)
    h = _tc_encode(x, p["gnn_enc"]["W"], row(p["gnn_enc"]["b"]))
    gsum = None
    for lp in p["gnn_layers"]:
        mp = _sc_segsum(h, src2, dst2, zrows)
        h, gsum = _tc_layer(h, mp, lp["W"], row(lp["b"]))

    ef = _sc_ef(x.reshape(-1), src, dst).reshape(4, N_EDGES)

    s_pad, mm = _tc_reduce(SCB, ef)
    AT, W24, c24T = _tc_prep(
        s_pad, p["scb1_a"]["W"], row(p["scb1_a"]["b"]),
        p["scb1_b"]["W"], row(p["scb1_b"]["b"]),
        p["scb2_a"]["W"][:64], row(p["scb2_a"]["b"]),
        p["scb2_b"]["W"], p["scb4_a"]["W"],
        row(p["scb2_b"]["b"]), row(p["scb4_a"]["b"]))
    T = _tc_edge(SCB, ef, AT, p["scb2_a"]["W"][64:], W24, c24T)

    return _tc_final(
        gsum, T, mm, p["scb4_b"]["W"], row(p["scb4_b"]["b"]),
        p["fin_a"]["W"], row(p["fin_a"]["b"]),
        p["fin_b"]["W"], row(p["fin_b"]["b"]),
        p["feat_a"]["W"], row(p["feat_a"]["b"]),
        p["feat_b"]["W"], row(p["feat_b"]["b"]),
        p["feat_c"]["W"], row(p["feat_c"]["b"]))
